# baseline (speedup 1.0000x reference)
import functools

import jax
import jax.numpy as jnp
from jax import lax
from jax.experimental import pallas as pl
from jax.experimental.pallas import tpu as pltpu
from jax.experimental.pallas import tpu_sc as plsc

D_MODEL = 128
CHUNK = 128
GROUP = 2
PAD_S = 56


def _add_kernel(a_ref, b_ref, o_ref):
    o_ref[...] = a_ref[...] + b_ref[...]


def _sum_tables(a, b):
    n, d = a.shape
    blk = 2000
    grid = n // blk
    return pl.pallas_call(
        _add_kernel,
        out_shape=jax.ShapeDtypeStruct((n, d), a.dtype),
        grid=(grid,),
        in_specs=[
            pl.BlockSpec((blk, d), lambda i: (i, 0)),
            pl.BlockSpec((blk, d), lambda i: (i, 0)),
        ],
        out_specs=pl.BlockSpec((blk, d), lambda i: (i, 0)),
    )(a, b)


@functools.lru_cache(maxsize=None)
def _make_gather(n_rows, chunk, group, d):
    info = plsc.get_sparse_core_info()
    nc, ns = info.num_cores, info.num_subcores
    nw = nc * ns
    per_w = n_rows // nw
    n_chunks = per_w // chunk
    n_groups = n_chunks // group
    grows = group * chunk

    mesh = plsc.VectorSubcoreMesh(core_axis_name="c", subcore_axis_name="s")

    @functools.partial(
        pl.kernel,
        out_type=jax.ShapeDtypeStruct((n_rows, d), jnp.float32),
        mesh=mesh,
        scratch_types=[
            pltpu.VMEM((n_chunks, chunk), jnp.int32),
            pltpu.VMEM((grows, d), jnp.float32),
            pltpu.VMEM((grows, d), jnp.float32),
            pltpu.SemaphoreType.DMA,
            pltpu.SemaphoreType.DMA,
            pltpu.SemaphoreType.DMA,
            pltpu.SemaphoreType.DMA,
        ],
    )
    def gather_kernel(x_hbm, tab_hbm, out_hbm, idx_v, buf0, buf1,
                      gsem0, gsem1, wsem0, wsem1):
        wid = lax.axis_index("s") * nc + lax.axis_index("c")
        row0 = wid * per_w
        pltpu.sync_copy(x_hbm.at[wid], idx_v)

        bufs = (buf0, buf1)
        gsems = (gsem0, gsem1)
        wsems = (wsem0, wsem1)

        def fire_group(g, slot):
            for k in range(group):
                pltpu.async_copy(
                    tab_hbm.at[idx_v.at[g * group + k]],
                    bufs[slot].at[pl.ds(k * chunk, chunk)],
                    gsems[slot],
                )

        def drain_gathers(slot):
            pltpu.make_async_copy(
                tab_hbm.at[pl.ds(0, grows)], bufs[slot], gsems[slot]
            ).wait()

        def fire_write(g, slot):
            pltpu.async_copy(
                bufs[slot],
                out_hbm.at[pl.ds(row0 + g * grows, grows)],
                wsems[slot],
            )

        def drain_write(slot):
            pltpu.make_async_copy(
                bufs[slot], out_hbm.at[pl.ds(row0, grows)], wsems[slot]
            ).wait()

        fire_group(0, 0)
        fire_group(1, 1)

        def body(i, carry):
            g0 = 2 * i
            drain_gathers(0)
            fire_write(g0, 0)
            drain_gathers(1)
            fire_write(g0 + 1, 1)

            @pl.when(i < n_groups // 2 - 1)
            def _refill():
                drain_write(0)
                fire_group(g0 + 2, 0)
                drain_write(1)
                fire_group(g0 + 3, 1)

            return carry

        lax.fori_loop(0, n_groups // 2, body, 0)
        drain_write(0)
        drain_write(1)

    return gather_kernel


def kernel(x, token_table, pos_table):
    summed = _sum_tables(token_table, pos_table)
    b, s = x.shape
    info = plsc.get_sparse_core_info()
    nw = info.num_cores * info.num_subcores
    xp = jnp.pad(x.astype(jnp.int32), ((0, 0), (0, PAD_S - s)))
    n_rows = b * PAD_S
    x3d = xp.reshape(nw, n_rows // nw // CHUNK, CHUNK)
    out = _make_gather(n_rows, CHUNK, GROUP, D_MODEL)(x3d, summed)
    return out.reshape(b, PAD_S, D_MODEL)[:, :s, :]
